# Initial kernel scaffold; baseline (speedup 1.0000x reference)
import jax
import jax.numpy as jnp
from jax.experimental import pallas as pl


def kernel(x, edge_index, batch, params):
    raise NotImplementedError("write your pallas kernel here")


# speedup vs baseline: 41.2529x; 41.2529x over previous
import functools

import jax
import jax.numpy as jnp
import numpy as np
from jax import lax
from jax.experimental import pallas as pl
from jax.experimental.pallas import tpu as pltpu
from jax.experimental.pallas import tpu_sc as plsc

N = 10000
HID = 128
H = 8
C = 16
NL = 6
VOCAB = 100

NC, NS, L = 2, 16, 16
NW = NC * NS

N_PAD = 10240
NB = N_PAD // 256
DUMMY = N

E_TOT = 320000 + N
K = 96
EPW = 10368
E_PAD = NW * EPW
NBLK = EPW // K

ACC_W = 144
RPS = N_PAD // NS
ZR = 16


def _sc_edge_body(src_hbm, dst_hbm, hh_hbm, aall_hbm, acc_hbm,
                  srcv, dstv, sbuf, dbuf, hbuf, mbuf, wflat, zbuf, acc_sh,
                  sem_h, sem_s, sem_d, sem_m):
    cid = lax.axis_index("c")
    sid = lax.axis_index("s")
    wid = sid * NC + cid

    def zr(i, _):
        zbuf[i // (ACC_W // 16), pl.ds((i % (ACC_W // 16)) * 16, 16)] = (
            jnp.zeros((16,), jnp.float32))
        return 0
    lax.fori_loop(0, ZR * (ACC_W // 16), zr, 0)

    def zc(k, _):
        pltpu.sync_copy(zbuf, acc_sh.at[pl.ds(sid * RPS + k * ZR, ZR)])
        return 0
    lax.fori_loop(0, RPS // ZR, zc, 0)
    plsc.subcore_barrier()

    lane = lax.iota(jnp.int32, 16)
    halfv = jnp.bitwise_and(lax.shift_right_logical(lane, 3), 1)
    col_s = jnp.bitwise_and(lane, 7)
    col_d = col_s + 8
    izero = jnp.zeros((16,), jnp.int32)
    zero16 = jnp.zeros((16,), jnp.float32)
    below8 = lane < 8

    def block(b, _):
        base = wid * EPW + b * K
        pltpu.sync_copy(src_hbm.at[pl.ds(base, K)], srcv)
        pltpu.sync_copy(dst_hbm.at[pl.ds(base, K)], dstv)
        ch = pltpu.async_copy(hh_hbm.at[srcv], hbuf, sem_h)
        cs = pltpu.async_copy(aall_hbm.at[srcv], sbuf, sem_s)
        cd = pltpu.async_copy(aall_hbm.at[dstv], dbuf, sem_d)
        cs.wait()
        cd.wait()
        ch.wait()

        def grp(j, _):
            r0 = 2 * j
            ridx = r0 + halfv
            sv = plsc.load_gather(sbuf, [ridx, col_s])
            dv = plsc.load_gather(dbuf, [ridx, col_d])
            a = sv + dv
            a = jnp.where(a > 0, a, 0.2 * a)
            w = jnp.exp(a)
            wflat[pl.ds(16 * j, 16)] = w
            w0 = plsc.load_gather(wflat, [16 * j + col_s])
            w1 = plsc.load_gather(wflat, [16 * j + 8 + col_s])
            mbuf[r0, pl.ds(HID, 16)] = jnp.where(below8, w0, zero16)
            mbuf[r0 + 1, pl.ds(HID, 16)] = jnp.where(below8, w1, zero16)
            for t in range(16):
                r = r0 + (t // 8)
                seg = pl.ds((t % 8) * 16, 16)
                wv = plsc.load_gather(wflat, [izero + (16 * j + t)])
                mbuf[r, seg] = hbuf[r, seg] * wv
            return 0
        lax.fori_loop(0, K // 2, grp, 0)

        cm = pltpu.async_copy(mbuf, acc_sh.at[dstv], sem_m, add=True)
        cm.wait()
        return 0
    lax.fori_loop(0, NBLK, block, 0)

    plsc.subcore_barrier()

    def co(k, _):
        row = sid * RPS + k * ZR
        pltpu.sync_copy(acc_sh.at[pl.ds(row, ZR)],
                        acc_hbm.at[cid, pl.ds(row, ZR)])
        return 0
    lax.fori_loop(0, RPS // ZR, co, 0)


def _sc_edge(hh, a_all, srcp, dstp):
    mesh = plsc.VectorSubcoreMesh(core_axis_name="c", subcore_axis_name="s",
                                  num_cores=NC, num_subcores=NS)
    f = pl.kernel(
        _sc_edge_body,
        out_type=jax.ShapeDtypeStruct((NC, N_PAD, ACC_W), jnp.float32),
        mesh=mesh,
        scratch_types=[
            pltpu.VMEM((K,), jnp.int32),
            pltpu.VMEM((K,), jnp.int32),
            pltpu.VMEM((K, 16), jnp.float32),
            pltpu.VMEM((K, 16), jnp.float32),
            pltpu.VMEM((K, HID), jnp.float32),
            pltpu.VMEM((K, ACC_W), jnp.float32),
            pltpu.VMEM((K * H,), jnp.float32),
            pltpu.VMEM((ZR, ACC_W), jnp.float32),
            pltpu.VMEM_SHARED((N_PAD, ACC_W), jnp.float32),
            pltpu.SemaphoreType.DMA,
            pltpu.SemaphoreType.DMA,
            pltpu.SemaphoreType.DMA,
            pltpu.SemaphoreType.DMA,
        ],
        name="gat_edge_sc",
        compiler_params=pltpu.CompilerParams(use_tc_tiling_on_sc=False,
                                             needs_layout_passes=False),
    )
    return f(srcp, dstp, hh, a_all)


def _ln_relu(out, g, b):
    mu = jnp.mean(out, axis=-1, keepdims=True)
    var = jnp.mean((out - mu) ** 2, axis=-1, keepdims=True)
    h = (out - mu) / jnp.sqrt(var + 1e-5) * g + b
    return jnp.maximum(h, 0.0)


def _epilogue(acc_ref, bias_ref, g_ref, b_ref):
    acc = acc_ref[0] + acc_ref[1]
    msg = acc[:, :HID].reshape(256, H, C)
    den = acc[:, HID:HID + H]
    out = (msg / (den + 1e-16)[:, :, None]).reshape(256, HID)
    return _ln_relu(out + bias_ref[...], g_ref[...], b_ref[...])


def _embed_body(x_ref, emb_ref, W_ref, A_ref, h_ref, hh_ref, a_ref):
    xb = x_ref[0, 0, :]
    ohT = (xb[None, :] == lax.broadcasted_iota(jnp.int32, (HID, 256), 0))
    h = lax.dot_general(ohT.astype(jnp.float32), emb_ref[...],
                        (((0,), (0,)), ((), ())),
                        preferred_element_type=jnp.float32)
    h_ref[...] = h
    hh = jnp.dot(h, W_ref[...], preferred_element_type=jnp.float32)
    hh_ref[...] = hh
    a_ref[...] = jnp.dot(hh, A_ref[...], preferred_element_type=jnp.float32)


def _embed(xp3, emb_pad, W0, A0):
    return pl.pallas_call(
        _embed_body,
        grid=(NB,),
        in_specs=[
            pl.BlockSpec((1, 1, 256), lambda i: (i, 0, 0)),
            pl.BlockSpec((HID, HID), lambda i: (0, 0)),
            pl.BlockSpec((HID, HID), lambda i: (0, 0)),
            pl.BlockSpec((HID, 16), lambda i: (0, 0)),
        ],
        out_specs=[
            pl.BlockSpec((256, HID), lambda i: (i, 0)),
            pl.BlockSpec((256, HID), lambda i: (i, 0)),
            pl.BlockSpec((256, 16), lambda i: (i, 0)),
        ],
        out_shape=[
            jax.ShapeDtypeStruct((N_PAD, HID), jnp.float32),
            jax.ShapeDtypeStruct((N_PAD, HID), jnp.float32),
            jax.ShapeDtypeStruct((N_PAD, 16), jnp.float32),
        ],
        name="gat_embed_tc",
    )(xp3, emb_pad, W0, A0)


def _mid_body(first, acc_ref, res_ref, bias_ref, g_ref, b_ref, W_ref, A_ref,
              h_ref, hh_ref, a_ref):
    h = _epilogue(acc_ref, bias_ref, g_ref, b_ref)
    if not first:
        h = h + res_ref[...]
    h_ref[...] = h
    hh = jnp.dot(h, W_ref[...], preferred_element_type=jnp.float32)
    hh_ref[...] = hh
    a_ref[...] = jnp.dot(hh, A_ref[...], preferred_element_type=jnp.float32)


def _mid(acc2, res, bias, g, b, Wn, An, first):
    return pl.pallas_call(
        functools.partial(_mid_body, first),
        grid=(NB,),
        in_specs=[
            pl.BlockSpec((NC, 256, ACC_W), lambda i: (0, i, 0)),
            pl.BlockSpec((256, HID), lambda i: (i, 0)),
            pl.BlockSpec((1, HID), lambda i: (0, 0)),
            pl.BlockSpec((1, HID), lambda i: (0, 0)),
            pl.BlockSpec((1, HID), lambda i: (0, 0)),
            pl.BlockSpec((HID, HID), lambda i: (0, 0)),
            pl.BlockSpec((HID, 16), lambda i: (0, 0)),
        ],
        out_specs=[
            pl.BlockSpec((256, HID), lambda i: (i, 0)),
            pl.BlockSpec((256, HID), lambda i: (i, 0)),
            pl.BlockSpec((256, 16), lambda i: (i, 0)),
        ],
        out_shape=[
            jax.ShapeDtypeStruct((N_PAD, HID), jnp.float32),
            jax.ShapeDtypeStruct((N_PAD, HID), jnp.float32),
            jax.ShapeDtypeStruct((N_PAD, 16), jnp.float32),
        ],
        name="gat_mid_tc",
    )(acc2, res, bias, g, b, Wn, An)


def _final_body(acc_ref, res_ref, bias_ref, g_ref, b_ref,
                W1_ref, b1_ref, W2_ref, b2_ref, W3_ref, b3_ref, y_ref):
    h = _epilogue(acc_ref, bias_ref, g_ref, b_ref) + res_ref[...]
    h1 = jnp.maximum(
        jnp.dot(h, W1_ref[...], preferred_element_type=jnp.float32)
        + b1_ref[...], 0.0)
    h2 = jnp.maximum(
        jnp.dot(h1, W2_ref[...], preferred_element_type=jnp.float32)
        + b2_ref[...], 0.0)
    y_ref[...] = (jnp.dot(h2, W3_ref[...], preferred_element_type=jnp.float32)
                  + b3_ref[...])


def _final(acc2, res, bias, g, b, W1, b1, W2, b2, W3p, b3p):
    return pl.pallas_call(
        _final_body,
        grid=(NB,),
        in_specs=[
            pl.BlockSpec((NC, 256, ACC_W), lambda i: (0, i, 0)),
            pl.BlockSpec((256, HID), lambda i: (i, 0)),
            pl.BlockSpec((1, HID), lambda i: (0, 0)),
            pl.BlockSpec((1, HID), lambda i: (0, 0)),
            pl.BlockSpec((1, HID), lambda i: (0, 0)),
            pl.BlockSpec((HID, 2 * HID), lambda i: (0, 0)),
            pl.BlockSpec((1, 2 * HID), lambda i: (0, 0)),
            pl.BlockSpec((2 * HID, HID), lambda i: (0, 0)),
            pl.BlockSpec((1, HID), lambda i: (0, 0)),
            pl.BlockSpec((HID, HID), lambda i: (0, 0)),
            pl.BlockSpec((1, HID), lambda i: (0, 0)),
        ],
        out_specs=[pl.BlockSpec((256, HID), lambda i: (i, 0))],
        out_shape=[jax.ShapeDtypeStruct((N_PAD, HID), jnp.float32)],
        name="gat_final_tc",
    )(acc2, res, bias, g, b, W1, b1, W2, b2, W3p, b3p)[0]


def kernel(x, edge_index, batch, params):
    del batch
    f32 = jnp.float32
    i32 = jnp.int32

    xp = jnp.zeros((N_PAD,), i32).at[:N].set(x.astype(i32))
    xp3 = xp.reshape(NB, 1, 256)

    emb_pad = jnp.zeros((HID, HID), f32).at[:VOCAB].set(params["emb"])

    loop = jnp.arange(N, dtype=i32)
    pad = jnp.full((E_PAD - E_TOT,), DUMMY, i32)
    srcp = jnp.concatenate([edge_index[0].astype(i32), loop, pad])
    dstp = jnp.concatenate([edge_index[1].astype(i32), loop, pad])

    rows = np.arange(HID)
    heads = rows // C
    Amats, Ws, biases, gs, bs = [], [], [], [], []
    for p in params["layers"]:
        A = jnp.zeros((HID, 16), f32)
        A = A.at[rows, heads].set(p["att_src"].reshape(HID))
        A = A.at[rows, heads + 8].set(p["att_dst"].reshape(HID))
        Amats.append(A)
        Ws.append(p["W"])
        biases.append(p["bias"].reshape(1, HID))
        gs.append(p["ln_g"].reshape(1, HID))
        bs.append(p["ln_b"].reshape(1, HID))

    h, hh, a_all = _embed(xp3, emb_pad, Ws[0], Amats[0])
    for i in range(NL):
        acc2 = _sc_edge(hh, a_all, srcp, dstp)
        if i < NL - 1:
            h, hh, a_all = _mid(acc2, h, biases[i], gs[i], bs[i],
                                Ws[i + 1], Amats[i + 1], first=(i == 0))
        else:
            W3p = jnp.zeros((HID, HID), f32).at[:, :3].set(params["W3"])
            b3p = jnp.zeros((1, HID), f32).at[0, :3].set(params["b3"])
            y = _final(acc2, h, biases[i], gs[i], bs[i],
                       params["W1"], params["b1"].reshape(1, 2 * HID),
                       params["W2"], params["b2"].reshape(1, HID),
                       W3p, b3p)
    return y[:N, :3]


# speedup vs baseline: 52.2503x; 1.2666x over previous
import functools

import jax
import jax.numpy as jnp
import numpy as np
from jax import lax
from jax.experimental import pallas as pl
from jax.experimental.pallas import tpu as pltpu
from jax.experimental.pallas import tpu_sc as plsc

N = 10000
HID = 128
H = 8
C = 16
NL = 6
VOCAB = 100

NC, NS, L = 2, 16, 16
NW = NC * NS

N_PAD = 10240
NB = N_PAD // 256
DUMMY = N

E_TOT = 320000 + N
K = 96
EPW = 10368
E_PAD = NW * EPW
NBLK = EPW // K

ACC_W = 144
RPS = N_PAD // NS
ZR = 16


NG = NBLK // 2


def _sc_edge_body(src_hbm, dst_hbm, hh_hbm, aall_hbm, accm_hbm, accd_hbm,
                  srcv0, dstv0, srcv1, dstv1, sbuf0, dbuf0, sbuf1, dbuf1,
                  hbuf0, hbuf1, wsc0, wsc1, wflat, zbuf, zdbuf,
                  accm_sh, accd_sh,
                  sem_h0, sem_s0, sem_d0, sem_m0, sem_w0,
                  sem_h1, sem_s1, sem_d1, sem_m1, sem_w1):
    cid = lax.axis_index("c")
    sid = lax.axis_index("s")
    wid = sid * NC + cid

    def zr(i, _):
        zbuf[i // (HID // 16), pl.ds((i % (HID // 16)) * 16, 16)] = (
            jnp.zeros((16,), jnp.float32))
        return 0
    lax.fori_loop(0, ZR * (HID // 16), zr, 0)

    def zrd(i, _):
        zdbuf[i, pl.ds(0, 16)] = jnp.zeros((16,), jnp.float32)
        return 0
    lax.fori_loop(0, ZR, zrd, 0)

    def zc(k, _):
        row = sid * RPS + k * ZR
        pltpu.sync_copy(zbuf, accm_sh.at[pl.ds(row, ZR)])
        pltpu.sync_copy(zdbuf, accd_sh.at[pl.ds(row, ZR)])
        return 0
    lax.fori_loop(0, RPS // ZR, zc, 0)
    plsc.subcore_barrier()

    lane = lax.iota(jnp.int32, 16)
    halfv = jnp.bitwise_and(lax.shift_right_logical(lane, 3), 1)
    col_s = jnp.bitwise_and(lane, 7)
    col_d = col_s + 8
    izero = jnp.zeros((16,), jnp.int32)
    zero16 = jnp.zeros((16,), jnp.float32)
    below8 = lane < 8
    ebase = wid * EPW

    bufs = ((srcv0, dstv0, sbuf0, dbuf0, hbuf0, wsc0,
             sem_h0, sem_s0, sem_d0, sem_m0, sem_w0),
            (srcv1, dstv1, sbuf1, dbuf1, hbuf1, wsc1,
             sem_h1, sem_s1, sem_d1, sem_m1, sem_w1))

    def idx_load(b, buf):
        srcv, dstv = buf[0], buf[1]
        base = ebase + b * K
        pltpu.sync_copy(src_hbm.at[pl.ds(base, K)], srcv)
        pltpu.sync_copy(dst_hbm.at[pl.ds(base, K)], dstv)

    def gather_start(buf):
        srcv, sbuf, dbuf, hbuf = buf[0], buf[2], buf[3], buf[4]
        dstv = buf[1]
        pltpu.async_copy(hh_hbm.at[srcv], hbuf, buf[6])
        pltpu.async_copy(aall_hbm.at[srcv], sbuf, buf[7])
        pltpu.async_copy(aall_hbm.at[dstv], dbuf, buf[8])

    def gather_wait(buf):
        srcv, dstv, sbuf, dbuf, hbuf = buf[0], buf[1], buf[2], buf[3], buf[4]
        pltpu.make_async_copy(hh_hbm.at[srcv], hbuf, buf[6]).wait()
        pltpu.make_async_copy(aall_hbm.at[srcv], sbuf, buf[7]).wait()
        pltpu.make_async_copy(aall_hbm.at[dstv], dbuf, buf[8]).wait()

    def scatter_start(buf):
        dstv, hbuf, wsc = buf[1], buf[4], buf[5]
        pltpu.async_copy(hbuf, accm_sh.at[dstv], buf[9], add=True)
        pltpu.async_copy(wsc, accd_sh.at[dstv], buf[10], add=True)

    def scatter_wait(buf):
        dstv, hbuf, wsc = buf[1], buf[4], buf[5]
        pltpu.make_async_copy(hbuf, accm_sh.at[dstv], buf[9]).wait()
        pltpu.make_async_copy(wsc, accd_sh.at[dstv], buf[10]).wait()

    def compute(buf):
        sbuf, dbuf, hbuf, wsc = buf[2], buf[3], buf[4], buf[5]

        def grp(j, _):
            r0 = 2 * j
            ridx = r0 + halfv
            sv = plsc.load_gather(sbuf, [ridx, col_s])
            dv = plsc.load_gather(dbuf, [ridx, col_d])
            a = sv + dv
            a = jnp.where(a > 0, a, 0.2 * a)
            w = jnp.exp(a)
            wflat[pl.ds(16 * j, 16)] = w
            w0 = plsc.load_gather(wflat, [16 * j + col_s])
            w1 = plsc.load_gather(wflat, [16 * j + 8 + col_s])
            wsc[r0, pl.ds(0, 16)] = jnp.where(below8, w0, zero16)
            wsc[r0 + 1, pl.ds(0, 16)] = jnp.where(below8, w1, zero16)
            for t in range(16):
                r = r0 + (t // 8)
                seg = pl.ds((t % 8) * 16, 16)
                wv = plsc.load_gather(wflat, [izero + (16 * j + t)])
                hbuf[r, seg] = hbuf[r, seg] * wv
            return 0
        lax.fori_loop(0, K // 2, grp, 0)

    idx_load(0, bufs[0])
    gather_start(bufs[0])

    def pair(g, _):
        @pl.when(g > 0)
        def _():
            scatter_wait(bufs[1])
        idx_load(2 * g + 1, bufs[1])
        gather_start(bufs[1])
        gather_wait(bufs[0])
        compute(bufs[0])
        scatter_start(bufs[0])
        scatter_wait(bufs[0])
        @pl.when(g < NG - 1)
        def _():
            idx_load(2 * g + 2, bufs[0])
            gather_start(bufs[0])
        gather_wait(bufs[1])
        compute(bufs[1])
        scatter_start(bufs[1])
        return 0
    lax.fori_loop(0, NG, pair, 0)
    scatter_wait(bufs[1])

    plsc.subcore_barrier()

    def co(k, _):
        row = sid * RPS + k * ZR
        pltpu.sync_copy(accm_sh.at[pl.ds(row, ZR)],
                        accm_hbm.at[cid, pl.ds(row, ZR)])
        pltpu.sync_copy(accd_sh.at[pl.ds(row, ZR)],
                        accd_hbm.at[cid, pl.ds(row, ZR)])
        return 0
    lax.fori_loop(0, RPS // ZR, co, 0)


def _sc_edge(hh, a_all, srcp, dstp):
    mesh = plsc.VectorSubcoreMesh(core_axis_name="c", subcore_axis_name="s",
                                  num_cores=NC, num_subcores=NS)
    f = pl.kernel(
        _sc_edge_body,
        out_type=[
            jax.ShapeDtypeStruct((NC, N_PAD, HID), jnp.float32),
            jax.ShapeDtypeStruct((NC, N_PAD, 16), jnp.float32),
        ],
        mesh=mesh,
        scratch_types=[
            pltpu.VMEM((K,), jnp.int32),
            pltpu.VMEM((K,), jnp.int32),
            pltpu.VMEM((K,), jnp.int32),
            pltpu.VMEM((K,), jnp.int32),
            pltpu.VMEM((K, 16), jnp.float32),
            pltpu.VMEM((K, 16), jnp.float32),
            pltpu.VMEM((K, 16), jnp.float32),
            pltpu.VMEM((K, 16), jnp.float32),
            pltpu.VMEM((K, HID), jnp.float32),
            pltpu.VMEM((K, HID), jnp.float32),
            pltpu.VMEM((K, 16), jnp.float32),
            pltpu.VMEM((K, 16), jnp.float32),
            pltpu.VMEM((K * H,), jnp.float32),
            pltpu.VMEM((ZR, HID), jnp.float32),
            pltpu.VMEM((ZR, 16), jnp.float32),
            pltpu.VMEM_SHARED((N_PAD, HID), jnp.float32),
            pltpu.VMEM_SHARED((N_PAD, 16), jnp.float32),
            pltpu.SemaphoreType.DMA,
            pltpu.SemaphoreType.DMA,
            pltpu.SemaphoreType.DMA,
            pltpu.SemaphoreType.DMA,
            pltpu.SemaphoreType.DMA,
            pltpu.SemaphoreType.DMA,
            pltpu.SemaphoreType.DMA,
            pltpu.SemaphoreType.DMA,
            pltpu.SemaphoreType.DMA,
            pltpu.SemaphoreType.DMA,
        ],
        name="gat_edge_sc",
        compiler_params=pltpu.CompilerParams(use_tc_tiling_on_sc=False,
                                             needs_layout_passes=False),
    )
    return f(srcp, dstp, hh, a_all)


def _ln_relu(out, g, b):
    mu = jnp.mean(out, axis=-1, keepdims=True)
    var = jnp.mean((out - mu) ** 2, axis=-1, keepdims=True)
    h = (out - mu) / jnp.sqrt(var + 1e-5) * g + b
    return jnp.maximum(h, 0.0)


def _epilogue(accm_ref, accd_ref, bias_ref, g_ref, b_ref):
    msg = (accm_ref[0] + accm_ref[1]).reshape(256, H, C)
    accd = accd_ref[0] + accd_ref[1]
    den = accd[:, :H]
    out = (msg / (den + 1e-16)[:, :, None]).reshape(256, HID)
    return _ln_relu(out + bias_ref[...], g_ref[...], b_ref[...])


def _embed_body(x_ref, emb_ref, W_ref, A_ref, h_ref, hh_ref, a_ref):
    xb = x_ref[0, 0, :]
    ohT = (xb[None, :] == lax.broadcasted_iota(jnp.int32, (HID, 256), 0))
    h = lax.dot_general(ohT.astype(jnp.float32), emb_ref[...],
                        (((0,), (0,)), ((), ())),
                        preferred_element_type=jnp.float32)
    h_ref[...] = h
    hh = jnp.dot(h, W_ref[...], preferred_element_type=jnp.float32)
    hh_ref[...] = hh
    a_ref[...] = jnp.dot(hh, A_ref[...], preferred_element_type=jnp.float32)


def _embed(xp3, emb_pad, W0, A0):
    return pl.pallas_call(
        _embed_body,
        grid=(NB,),
        in_specs=[
            pl.BlockSpec((1, 1, 256), lambda i: (i, 0, 0)),
            pl.BlockSpec((HID, HID), lambda i: (0, 0)),
            pl.BlockSpec((HID, HID), lambda i: (0, 0)),
            pl.BlockSpec((HID, 16), lambda i: (0, 0)),
        ],
        out_specs=[
            pl.BlockSpec((256, HID), lambda i: (i, 0)),
            pl.BlockSpec((256, HID), lambda i: (i, 0)),
            pl.BlockSpec((256, 16), lambda i: (i, 0)),
        ],
        out_shape=[
            jax.ShapeDtypeStruct((N_PAD, HID), jnp.float32),
            jax.ShapeDtypeStruct((N_PAD, HID), jnp.float32),
            jax.ShapeDtypeStruct((N_PAD, 16), jnp.float32),
        ],
        name="gat_embed_tc",
    )(xp3, emb_pad, W0, A0)


def _mid_body(first, accm_ref, accd_ref, res_ref, bias_ref, g_ref, b_ref,
              W_ref, A_ref, h_ref, hh_ref, a_ref):
    h = _epilogue(accm_ref, accd_ref, bias_ref, g_ref, b_ref)
    if not first:
        h = h + res_ref[...]
    h_ref[...] = h
    hh = jnp.dot(h, W_ref[...], preferred_element_type=jnp.float32)
    hh_ref[...] = hh
    a_ref[...] = jnp.dot(hh, A_ref[...], preferred_element_type=jnp.float32)


def _mid(accm, accd, res, bias, g, b, Wn, An, first):
    return pl.pallas_call(
        functools.partial(_mid_body, first),
        grid=(NB,),
        in_specs=[
            pl.BlockSpec((NC, 256, HID), lambda i: (0, i, 0)),
            pl.BlockSpec((NC, 256, 16), lambda i: (0, i, 0)),
            pl.BlockSpec((256, HID), lambda i: (i, 0)),
            pl.BlockSpec((1, HID), lambda i: (0, 0)),
            pl.BlockSpec((1, HID), lambda i: (0, 0)),
            pl.BlockSpec((1, HID), lambda i: (0, 0)),
            pl.BlockSpec((HID, HID), lambda i: (0, 0)),
            pl.BlockSpec((HID, 16), lambda i: (0, 0)),
        ],
        out_specs=[
            pl.BlockSpec((256, HID), lambda i: (i, 0)),
            pl.BlockSpec((256, HID), lambda i: (i, 0)),
            pl.BlockSpec((256, 16), lambda i: (i, 0)),
        ],
        out_shape=[
            jax.ShapeDtypeStruct((N_PAD, HID), jnp.float32),
            jax.ShapeDtypeStruct((N_PAD, HID), jnp.float32),
            jax.ShapeDtypeStruct((N_PAD, 16), jnp.float32),
        ],
        name="gat_mid_tc",
    )(accm, accd, res, bias, g, b, Wn, An)


def _final_body(accm_ref, accd_ref, res_ref, bias_ref, g_ref, b_ref,
                W1_ref, b1_ref, W2_ref, b2_ref, W3_ref, b3_ref, y_ref):
    h = _epilogue(accm_ref, accd_ref, bias_ref, g_ref, b_ref) + res_ref[...]
    h1 = jnp.maximum(
        jnp.dot(h, W1_ref[...], preferred_element_type=jnp.float32)
        + b1_ref[...], 0.0)
    h2 = jnp.maximum(
        jnp.dot(h1, W2_ref[...], preferred_element_type=jnp.float32)
        + b2_ref[...], 0.0)
    y_ref[...] = (jnp.dot(h2, W3_ref[...], preferred_element_type=jnp.float32)
                  + b3_ref[...])


def _final(accm, accd, res, bias, g, b, W1, b1, W2, b2, W3p, b3p):
    return pl.pallas_call(
        _final_body,
        grid=(NB,),
        in_specs=[
            pl.BlockSpec((NC, 256, HID), lambda i: (0, i, 0)),
            pl.BlockSpec((NC, 256, 16), lambda i: (0, i, 0)),
            pl.BlockSpec((256, HID), lambda i: (i, 0)),
            pl.BlockSpec((1, HID), lambda i: (0, 0)),
            pl.BlockSpec((1, HID), lambda i: (0, 0)),
            pl.BlockSpec((1, HID), lambda i: (0, 0)),
            pl.BlockSpec((HID, 2 * HID), lambda i: (0, 0)),
            pl.BlockSpec((1, 2 * HID), lambda i: (0, 0)),
            pl.BlockSpec((2 * HID, HID), lambda i: (0, 0)),
            pl.BlockSpec((1, HID), lambda i: (0, 0)),
            pl.BlockSpec((HID, HID), lambda i: (0, 0)),
            pl.BlockSpec((1, HID), lambda i: (0, 0)),
        ],
        out_specs=[pl.BlockSpec((256, HID), lambda i: (i, 0))],
        out_shape=[jax.ShapeDtypeStruct((N_PAD, HID), jnp.float32)],
        name="gat_final_tc",
    )(accm, accd, res, bias, g, b, W1, b1, W2, b2, W3p, b3p)[0]


def kernel(x, edge_index, batch, params):
    del batch
    f32 = jnp.float32
    i32 = jnp.int32

    xp = jnp.zeros((N_PAD,), i32).at[:N].set(x.astype(i32))
    xp3 = xp.reshape(NB, 1, 256)

    emb_pad = jnp.zeros((HID, HID), f32).at[:VOCAB].set(params["emb"])

    loop = jnp.arange(N, dtype=i32)
    pad = jnp.full((E_PAD - E_TOT,), DUMMY, i32)
    srcp = jnp.concatenate([edge_index[0].astype(i32), loop, pad])
    dstp = jnp.concatenate([edge_index[1].astype(i32), loop, pad])

    rows = np.arange(HID)
    heads = rows // C
    Amats, Ws, biases, gs, bs = [], [], [], [], []
    for p in params["layers"]:
        A = jnp.zeros((HID, 16), f32)
        A = A.at[rows, heads].set(p["att_src"].reshape(HID))
        A = A.at[rows, heads + 8].set(p["att_dst"].reshape(HID))
        Amats.append(A)
        Ws.append(p["W"])
        biases.append(p["bias"].reshape(1, HID))
        gs.append(p["ln_g"].reshape(1, HID))
        bs.append(p["ln_b"].reshape(1, HID))

    h, hh, a_all = _embed(xp3, emb_pad, Ws[0], Amats[0])
    for i in range(NL):
        accm, accd = _sc_edge(hh, a_all, srcp, dstp)
        if i < NL - 1:
            h, hh, a_all = _mid(accm, accd, h, biases[i], gs[i], bs[i],
                                Ws[i + 1], Amats[i + 1], first=(i == 0))
        else:
            W3p = jnp.zeros((HID, HID), f32).at[:, :3].set(params["W3"])
            b3p = jnp.zeros((1, HID), f32).at[0, :3].set(params["b3"])
            y = _final(accm, accd, h, biases[i], gs[i], bs[i],
                       params["W1"], params["b1"].reshape(1, 2 * HID),
                       params["W2"], params["b2"].reshape(1, HID),
                       W3p, b3p)
    return y[:N, :3]


# speedup vs baseline: 102.5566x; 1.9628x over previous
import functools

import jax
import jax.numpy as jnp
import numpy as np
from jax import lax
from jax.experimental import pallas as pl
from jax.experimental.pallas import tpu as pltpu
from jax.experimental.pallas import tpu_sc as plsc

N = 10000
HID = 128
H = 8
C = 16
NL = 6
VOCAB = 100

NC, NS, L = 2, 16, 16
NW = NC * NS

N_PAD = 10240
NB = N_PAD // 256
DUMMY = N

E_TOT = 320000 + N
K = 96
EPW = 10368
E_PAD = NW * EPW
NBLK = EPW // K

ACC_W = 144
RPS = N_PAD // NS
ZR = 16


NG = NBLK // 2


def _sc_edge_body(src_hbm, dst_hbm, hh_hbm, aall_hbm, accm_hbm, accd_hbm,
                  srcv0, dstv0, srcv1, dstv1, sbuf0, dbuf0, sbuf1, dbuf1,
                  hbuf0, hbuf1, wsc0, wsc1, wflat, zbuf, zdbuf,
                  accm_sh, accd_sh,
                  sem_h0, sem_s0, sem_d0, sem_m0, sem_w0,
                  sem_h1, sem_s1, sem_d1, sem_m1, sem_w1):
    cid = lax.axis_index("c")
    sid = lax.axis_index("s")
    wid = sid * NC + cid

    def zr(i, _):
        zbuf[i // (HID // 16), pl.ds((i % (HID // 16)) * 16, 16)] = (
            jnp.zeros((16,), jnp.float32))
        return 0
    lax.fori_loop(0, ZR * (HID // 16), zr, 0)

    def zrd(i, _):
        zdbuf[i, pl.ds(0, 16)] = jnp.zeros((16,), jnp.float32)
        return 0
    lax.fori_loop(0, ZR, zrd, 0)

    def zc(k, _):
        row = sid * RPS + k * ZR
        pltpu.sync_copy(zbuf, accm_sh.at[pl.ds(row, ZR)])
        pltpu.sync_copy(zdbuf, accd_sh.at[pl.ds(row, ZR)])
        return 0
    lax.fori_loop(0, RPS // ZR, zc, 0)
    plsc.subcore_barrier()

    lane = lax.iota(jnp.int32, 16)
    halfv = jnp.bitwise_and(lax.shift_right_logical(lane, 3), 1)
    col_s = jnp.bitwise_and(lane, 7)
    col_d = col_s + 8
    izero = jnp.zeros((16,), jnp.int32)
    zero16 = jnp.zeros((16,), jnp.float32)
    below8 = lane < 8
    ebase = wid * EPW

    bufs = ((srcv0, dstv0, sbuf0, dbuf0, hbuf0, wsc0,
             sem_h0, sem_s0, sem_d0, sem_m0, sem_w0),
            (srcv1, dstv1, sbuf1, dbuf1, hbuf1, wsc1,
             sem_h1, sem_s1, sem_d1, sem_m1, sem_w1))

    def idx_load(b, buf):
        srcv, dstv = buf[0], buf[1]
        base = ebase + b * K
        pltpu.sync_copy(src_hbm.at[pl.ds(base, K)], srcv)
        pltpu.sync_copy(dst_hbm.at[pl.ds(base, K)], dstv)

    def gather_start(buf):
        srcv, sbuf, dbuf, hbuf = buf[0], buf[2], buf[3], buf[4]
        dstv = buf[1]
        pltpu.async_copy(hh_hbm.at[srcv], hbuf, buf[6])
        pltpu.async_copy(aall_hbm.at[srcv], sbuf, buf[7])
        pltpu.async_copy(aall_hbm.at[dstv], dbuf, buf[8])

    def gather_wait(buf):
        srcv, dstv, sbuf, dbuf, hbuf = buf[0], buf[1], buf[2], buf[3], buf[4]
        pltpu.make_async_copy(hh_hbm.at[srcv], hbuf, buf[6]).wait()
        pltpu.make_async_copy(aall_hbm.at[srcv], sbuf, buf[7]).wait()
        pltpu.make_async_copy(aall_hbm.at[dstv], dbuf, buf[8]).wait()

    def scatter_start(buf):
        dstv, hbuf, wsc = buf[1], buf[4], buf[5]
        pltpu.async_copy(hbuf, accm_sh.at[dstv], buf[9], add=True)
        pltpu.async_copy(wsc, accd_sh.at[dstv], buf[10], add=True)

    def scatter_wait(buf):
        dstv, hbuf, wsc = buf[1], buf[4], buf[5]
        pltpu.make_async_copy(hbuf, accm_sh.at[dstv], buf[9]).wait()
        pltpu.make_async_copy(wsc, accd_sh.at[dstv], buf[10]).wait()

    def compute(buf):
        sbuf, dbuf, hbuf, wsc = buf[2], buf[3], buf[4], buf[5]

        @plsc.parallel_loop(0, K // 2, unroll=2)
        def _(j):
            r0 = 2 * j
            ridx = r0 + halfv
            sv = plsc.load_gather(sbuf, [ridx, col_s])
            dv = plsc.load_gather(dbuf, [ridx, col_d])
            a = sv + dv
            a = jnp.where(a > 0, a, 0.2 * a)
            w = jnp.exp(a)
            wflat[pl.ds(16 * j, 16)] = w
            w1 = plsc.load_gather(wflat, [16 * j + 8 + col_s])
            wsc[r0, pl.ds(0, 16)] = jnp.where(below8, w, zero16)
            wsc[r0 + 1, pl.ds(0, 16)] = jnp.where(below8, w1, zero16)

        @plsc.parallel_loop(0, K, unroll=2)
        def _(r):
            wrow = wsc[r, pl.ds(0, 16)]
            for t in range(H):
                seg = pl.ds(t * C, 16)
                hbuf[r, seg] = hbuf[r, seg] * (zero16 + wrow[t])

    idx_load(0, bufs[0])
    gather_start(bufs[0])

    def pair(g, _):
        @pl.when(g > 0)
        def _():
            scatter_wait(bufs[1])
        idx_load(2 * g + 1, bufs[1])
        gather_start(bufs[1])
        gather_wait(bufs[0])
        compute(bufs[0])
        scatter_start(bufs[0])
        scatter_wait(bufs[0])
        @pl.when(g < NG - 1)
        def _():
            idx_load(2 * g + 2, bufs[0])
            gather_start(bufs[0])
        gather_wait(bufs[1])
        compute(bufs[1])
        scatter_start(bufs[1])
        return 0
    lax.fori_loop(0, NG, pair, 0)
    scatter_wait(bufs[1])

    plsc.subcore_barrier()

    def co(k, _):
        row = sid * RPS + k * ZR
        pltpu.sync_copy(accm_sh.at[pl.ds(row, ZR)],
                        accm_hbm.at[cid, pl.ds(row, ZR)])
        pltpu.sync_copy(accd_sh.at[pl.ds(row, ZR)],
                        accd_hbm.at[cid, pl.ds(row, ZR)])
        return 0
    lax.fori_loop(0, RPS // ZR, co, 0)


def _sc_edge(hh, a_all, srcp, dstp):
    mesh = plsc.VectorSubcoreMesh(core_axis_name="c", subcore_axis_name="s",
                                  num_cores=NC, num_subcores=NS)
    f = pl.kernel(
        _sc_edge_body,
        out_type=[
            jax.ShapeDtypeStruct((NC, N_PAD, HID), jnp.float32),
            jax.ShapeDtypeStruct((NC, N_PAD, 16), jnp.float32),
        ],
        mesh=mesh,
        scratch_types=[
            pltpu.VMEM((K,), jnp.int32),
            pltpu.VMEM((K,), jnp.int32),
            pltpu.VMEM((K,), jnp.int32),
            pltpu.VMEM((K,), jnp.int32),
            pltpu.VMEM((K, 16), jnp.float32),
            pltpu.VMEM((K, 16), jnp.float32),
            pltpu.VMEM((K, 16), jnp.float32),
            pltpu.VMEM((K, 16), jnp.float32),
            pltpu.VMEM((K, HID), jnp.float32),
            pltpu.VMEM((K, HID), jnp.float32),
            pltpu.VMEM((K, 16), jnp.float32),
            pltpu.VMEM((K, 16), jnp.float32),
            pltpu.VMEM((K * H,), jnp.float32),
            pltpu.VMEM((ZR, HID), jnp.float32),
            pltpu.VMEM((ZR, 16), jnp.float32),
            pltpu.VMEM_SHARED((N_PAD, HID), jnp.float32),
            pltpu.VMEM_SHARED((N_PAD, 16), jnp.float32),
            pltpu.SemaphoreType.DMA,
            pltpu.SemaphoreType.DMA,
            pltpu.SemaphoreType.DMA,
            pltpu.SemaphoreType.DMA,
            pltpu.SemaphoreType.DMA,
            pltpu.SemaphoreType.DMA,
            pltpu.SemaphoreType.DMA,
            pltpu.SemaphoreType.DMA,
            pltpu.SemaphoreType.DMA,
            pltpu.SemaphoreType.DMA,
        ],
        name="gat_edge_sc",
        compiler_params=pltpu.CompilerParams(use_tc_tiling_on_sc=False,
                                             needs_layout_passes=False),
    )
    return f(srcp, dstp, hh, a_all)


def _ln_relu(out, g, b):
    mu = jnp.mean(out, axis=-1, keepdims=True)
    var = jnp.mean((out - mu) ** 2, axis=-1, keepdims=True)
    h = (out - mu) / jnp.sqrt(var + 1e-5) * g + b
    return jnp.maximum(h, 0.0)


def _epilogue(accm_ref, accd_ref, bias_ref, g_ref, b_ref):
    msg = (accm_ref[0] + accm_ref[1]).reshape(256, H, C)
    accd = accd_ref[0] + accd_ref[1]
    den = accd[:, :H]
    out = (msg / (den + 1e-16)[:, :, None]).reshape(256, HID)
    return _ln_relu(out + bias_ref[...], g_ref[...], b_ref[...])


def _embed_body(x_ref, emb_ref, W_ref, A_ref, h_ref, hh_ref, a_ref):
    xb = x_ref[0, 0, :]
    ohT = (xb[None, :] == lax.broadcasted_iota(jnp.int32, (HID, 256), 0))
    h = lax.dot_general(ohT.astype(jnp.float32), emb_ref[...],
                        (((0,), (0,)), ((), ())),
                        preferred_element_type=jnp.float32)
    h_ref[...] = h
    hh = jnp.dot(h, W_ref[...], preferred_element_type=jnp.float32)
    hh_ref[...] = hh
    a_ref[...] = jnp.dot(hh, A_ref[...], preferred_element_type=jnp.float32)


def _embed(xp3, emb_pad, W0, A0):
    return pl.pallas_call(
        _embed_body,
        grid=(NB,),
        in_specs=[
            pl.BlockSpec((1, 1, 256), lambda i: (i, 0, 0)),
            pl.BlockSpec((HID, HID), lambda i: (0, 0)),
            pl.BlockSpec((HID, HID), lambda i: (0, 0)),
            pl.BlockSpec((HID, 16), lambda i: (0, 0)),
        ],
        out_specs=[
            pl.BlockSpec((256, HID), lambda i: (i, 0)),
            pl.BlockSpec((256, HID), lambda i: (i, 0)),
            pl.BlockSpec((256, 16), lambda i: (i, 0)),
        ],
        out_shape=[
            jax.ShapeDtypeStruct((N_PAD, HID), jnp.float32),
            jax.ShapeDtypeStruct((N_PAD, HID), jnp.float32),
            jax.ShapeDtypeStruct((N_PAD, 16), jnp.float32),
        ],
        name="gat_embed_tc",
    )(xp3, emb_pad, W0, A0)


def _mid_body(first, accm_ref, accd_ref, res_ref, bias_ref, g_ref, b_ref,
              W_ref, A_ref, h_ref, hh_ref, a_ref):
    h = _epilogue(accm_ref, accd_ref, bias_ref, g_ref, b_ref)
    if not first:
        h = h + res_ref[...]
    h_ref[...] = h
    hh = jnp.dot(h, W_ref[...], preferred_element_type=jnp.float32)
    hh_ref[...] = hh
    a_ref[...] = jnp.dot(hh, A_ref[...], preferred_element_type=jnp.float32)


def _mid(accm, accd, res, bias, g, b, Wn, An, first):
    return pl.pallas_call(
        functools.partial(_mid_body, first),
        grid=(NB,),
        in_specs=[
            pl.BlockSpec((NC, 256, HID), lambda i: (0, i, 0)),
            pl.BlockSpec((NC, 256, 16), lambda i: (0, i, 0)),
            pl.BlockSpec((256, HID), lambda i: (i, 0)),
            pl.BlockSpec((1, HID), lambda i: (0, 0)),
            pl.BlockSpec((1, HID), lambda i: (0, 0)),
            pl.BlockSpec((1, HID), lambda i: (0, 0)),
            pl.BlockSpec((HID, HID), lambda i: (0, 0)),
            pl.BlockSpec((HID, 16), lambda i: (0, 0)),
        ],
        out_specs=[
            pl.BlockSpec((256, HID), lambda i: (i, 0)),
            pl.BlockSpec((256, HID), lambda i: (i, 0)),
            pl.BlockSpec((256, 16), lambda i: (i, 0)),
        ],
        out_shape=[
            jax.ShapeDtypeStruct((N_PAD, HID), jnp.float32),
            jax.ShapeDtypeStruct((N_PAD, HID), jnp.float32),
            jax.ShapeDtypeStruct((N_PAD, 16), jnp.float32),
        ],
        name="gat_mid_tc",
    )(accm, accd, res, bias, g, b, Wn, An)


def _final_body(accm_ref, accd_ref, res_ref, bias_ref, g_ref, b_ref,
                W1_ref, b1_ref, W2_ref, b2_ref, W3_ref, b3_ref, y_ref):
    h = _epilogue(accm_ref, accd_ref, bias_ref, g_ref, b_ref) + res_ref[...]
    h1 = jnp.maximum(
        jnp.dot(h, W1_ref[...], preferred_element_type=jnp.float32)
        + b1_ref[...], 0.0)
    h2 = jnp.maximum(
        jnp.dot(h1, W2_ref[...], preferred_element_type=jnp.float32)
        + b2_ref[...], 0.0)
    y_ref[...] = (jnp.dot(h2, W3_ref[...], preferred_element_type=jnp.float32)
                  + b3_ref[...])


def _final(accm, accd, res, bias, g, b, W1, b1, W2, b2, W3p, b3p):
    return pl.pallas_call(
        _final_body,
        grid=(NB,),
        in_specs=[
            pl.BlockSpec((NC, 256, HID), lambda i: (0, i, 0)),
            pl.BlockSpec((NC, 256, 16), lambda i: (0, i, 0)),
            pl.BlockSpec((256, HID), lambda i: (i, 0)),
            pl.BlockSpec((1, HID), lambda i: (0, 0)),
            pl.BlockSpec((1, HID), lambda i: (0, 0)),
            pl.BlockSpec((1, HID), lambda i: (0, 0)),
            pl.BlockSpec((HID, 2 * HID), lambda i: (0, 0)),
            pl.BlockSpec((1, 2 * HID), lambda i: (0, 0)),
            pl.BlockSpec((2 * HID, HID), lambda i: (0, 0)),
            pl.BlockSpec((1, HID), lambda i: (0, 0)),
            pl.BlockSpec((HID, HID), lambda i: (0, 0)),
            pl.BlockSpec((1, HID), lambda i: (0, 0)),
        ],
        out_specs=[pl.BlockSpec((256, HID), lambda i: (i, 0))],
        out_shape=[jax.ShapeDtypeStruct((N_PAD, HID), jnp.float32)],
        name="gat_final_tc",
    )(accm, accd, res, bias, g, b, W1, b1, W2, b2, W3p, b3p)[0]


def kernel(x, edge_index, batch, params):
    del batch
    f32 = jnp.float32
    i32 = jnp.int32

    xp = jnp.zeros((N_PAD,), i32).at[:N].set(x.astype(i32))
    xp3 = xp.reshape(NB, 1, 256)

    emb_pad = jnp.zeros((HID, HID), f32).at[:VOCAB].set(params["emb"])

    loop = jnp.arange(N, dtype=i32)
    pad = jnp.full((E_PAD - E_TOT,), DUMMY, i32)
    srcp = jnp.concatenate([edge_index[0].astype(i32), loop, pad])
    dstp = jnp.concatenate([edge_index[1].astype(i32), loop, pad])

    rows = np.arange(HID)
    heads = rows // C
    Amats, Ws, biases, gs, bs = [], [], [], [], []
    for p in params["layers"]:
        A = jnp.zeros((HID, 16), f32)
        A = A.at[rows, heads].set(p["att_src"].reshape(HID))
        A = A.at[rows, heads + 8].set(p["att_dst"].reshape(HID))
        Amats.append(A)
        Ws.append(p["W"])
        biases.append(p["bias"].reshape(1, HID))
        gs.append(p["ln_g"].reshape(1, HID))
        bs.append(p["ln_b"].reshape(1, HID))

    h, hh, a_all = _embed(xp3, emb_pad, Ws[0], Amats[0])
    for i in range(NL):
        accm, accd = _sc_edge(hh, a_all, srcp, dstp)
        if i < NL - 1:
            h, hh, a_all = _mid(accm, accd, h, biases[i], gs[i], bs[i],
                                Ws[i + 1], Amats[i + 1], first=(i == 0))
        else:
            W3p = jnp.zeros((HID, HID), f32).at[:, :3].set(params["W3"])
            b3p = jnp.zeros((1, HID), f32).at[0, :3].set(params["b3"])
            y = _final(accm, accd, h, biases[i], gs[i], bs[i],
                       params["W1"], params["b1"].reshape(1, 2 * HID),
                       params["W2"], params["b2"].reshape(1, HID),
                       W3p, b3p)
    return y[:N, :3]


# speedup vs baseline: 111.1005x; 1.0833x over previous
import functools

import jax
import jax.numpy as jnp
import numpy as np
from jax import lax
from jax.experimental import pallas as pl
from jax.experimental.pallas import tpu as pltpu
from jax.experimental.pallas import tpu_sc as plsc

N = 10000
HID = 128
H = 8
C = 16
NL = 6
VOCAB = 100

NC, NS, L = 2, 16, 16
NW = NC * NS

N_PAD = 10240
NB = N_PAD // 256
DUMMY = N

E_TOT = 320000 + N
K = 96
EPW = 10368
E_PAD = NW * EPW
NBLK = EPW // K

ACC_W = 144
RPS = N_PAD // NS
ZR = 16


NG = NBLK // 2


def _sc_edge_body(idx_hbm, hh_hbm, aall_hbm, accm_hbm, accd_hbm,
                  idxb0, idxb1, sbuf0, dbuf0, sbuf1, dbuf1,
                  hbuf0, hbuf1, wsc0, wsc1, wflat, zbuf, zdbuf,
                  accm_sh, accd_sh,
                  sem_h0, sem_s0, sem_d0, sem_m0, sem_w0,
                  sem_h1, sem_s1, sem_d1, sem_m1, sem_w1):
    cid = lax.axis_index("c")
    sid = lax.axis_index("s")
    wid = sid * NC + cid

    def zr(i, _):
        zbuf[i // (HID // 16), pl.ds((i % (HID // 16)) * 16, 16)] = (
            jnp.zeros((16,), jnp.float32))
        return 0
    lax.fori_loop(0, ZR * (HID // 16), zr, 0)

    def zrd(i, _):
        zdbuf[i, pl.ds(0, 16)] = jnp.zeros((16,), jnp.float32)
        return 0
    lax.fori_loop(0, ZR, zrd, 0)

    def zc(k, _):
        row = sid * RPS + k * ZR
        pltpu.sync_copy(zbuf, accm_sh.at[pl.ds(row, ZR)])
        pltpu.sync_copy(zdbuf, accd_sh.at[pl.ds(row, ZR)])
        return 0
    lax.fori_loop(0, RPS // ZR, zc, 0)
    plsc.subcore_barrier()

    lane = lax.iota(jnp.int32, 16)
    halfv = jnp.bitwise_and(lax.shift_right_logical(lane, 3), 1)
    col_s = jnp.bitwise_and(lane, 7)
    col_d = col_s + 8
    izero = jnp.zeros((16,), jnp.int32)
    zero16 = jnp.zeros((16,), jnp.float32)
    below8 = lane < 8
    bbase = wid * NBLK

    bufs = ((idxb0, None, sbuf0, dbuf0, hbuf0, wsc0,
             sem_h0, sem_s0, sem_d0, sem_m0, sem_w0),
            (idxb1, None, sbuf1, dbuf1, hbuf1, wsc1,
             sem_h1, sem_s1, sem_d1, sem_m1, sem_w1))

    def idx_load(b, buf):
        pltpu.sync_copy(idx_hbm.at[bbase + b], buf[0])

    def gather_start(buf):
        idxb, sbuf, dbuf, hbuf = buf[0], buf[2], buf[3], buf[4]
        pltpu.async_copy(hh_hbm.at[idxb.at[0]], hbuf, buf[6])
        pltpu.async_copy(aall_hbm.at[idxb.at[0]], sbuf, buf[7])
        pltpu.async_copy(aall_hbm.at[idxb.at[1]], dbuf, buf[8])

    def gather_wait(buf):
        idxb, sbuf, dbuf, hbuf = buf[0], buf[2], buf[3], buf[4]
        pltpu.make_async_copy(hh_hbm.at[idxb.at[0]], hbuf, buf[6]).wait()
        pltpu.make_async_copy(aall_hbm.at[idxb.at[0]], sbuf, buf[7]).wait()
        pltpu.make_async_copy(aall_hbm.at[idxb.at[1]], dbuf, buf[8]).wait()

    def scatter_start(buf):
        idxb, hbuf, wsc = buf[0], buf[4], buf[5]
        pltpu.async_copy(hbuf, accm_sh.at[idxb.at[1]], buf[9], add=True)
        pltpu.async_copy(wsc, accd_sh.at[idxb.at[1]], buf[10], add=True)

    def scatter_wait(buf):
        idxb, hbuf, wsc = buf[0], buf[4], buf[5]
        pltpu.make_async_copy(hbuf, accm_sh.at[idxb.at[1]], buf[9]).wait()
        pltpu.make_async_copy(wsc, accd_sh.at[idxb.at[1]], buf[10]).wait()

    def compute(buf):
        sbuf, dbuf, hbuf, wsc = buf[2], buf[3], buf[4], buf[5]

        @plsc.parallel_loop(0, K // 2, unroll=2)
        def _(j):
            r0 = 2 * j
            ridx = r0 + halfv
            sv = plsc.load_gather(sbuf, [ridx, col_s])
            dv = plsc.load_gather(dbuf, [ridx, col_d])
            a = sv + dv
            a = jnp.where(a > 0, a, 0.2 * a)
            w = jnp.exp(a)
            wflat[pl.ds(16 * j, 16)] = w
            w1 = plsc.load_gather(wflat, [16 * j + 8 + col_s])
            wsc[r0, pl.ds(0, 16)] = jnp.where(below8, w, zero16)
            wsc[r0 + 1, pl.ds(0, 16)] = jnp.where(below8, w1, zero16)

        @plsc.parallel_loop(0, K, unroll=2)
        def _(r):
            wrow = wsc[r, pl.ds(0, 16)]
            for t in range(H):
                seg = pl.ds(t * C, 16)
                hbuf[r, seg] = hbuf[r, seg] * (zero16 + wrow[t])

    idx_load(0, bufs[0])
    gather_start(bufs[0])

    def pair(g, _):
        @pl.when(g > 0)
        def _():
            scatter_wait(bufs[1])
        idx_load(2 * g + 1, bufs[1])
        gather_start(bufs[1])
        gather_wait(bufs[0])
        compute(bufs[0])
        scatter_start(bufs[0])
        scatter_wait(bufs[0])
        @pl.when(g < NG - 1)
        def _():
            idx_load(2 * g + 2, bufs[0])
            gather_start(bufs[0])
        gather_wait(bufs[1])
        compute(bufs[1])
        scatter_start(bufs[1])
        return 0
    lax.fori_loop(0, NG, pair, 0)
    scatter_wait(bufs[1])

    plsc.subcore_barrier()

    def co(k, _):
        row = sid * RPS + k * ZR
        pltpu.sync_copy(accm_sh.at[pl.ds(row, ZR)],
                        accm_hbm.at[cid, pl.ds(row, ZR)])
        pltpu.sync_copy(accd_sh.at[pl.ds(row, ZR)],
                        accd_hbm.at[cid, pl.ds(row, ZR)])
        return 0
    lax.fori_loop(0, RPS // ZR, co, 0)


def _sc_edge(hh, a_all, idx2):
    mesh = plsc.VectorSubcoreMesh(core_axis_name="c", subcore_axis_name="s",
                                  num_cores=NC, num_subcores=NS)
    f = pl.kernel(
        _sc_edge_body,
        out_type=[
            jax.ShapeDtypeStruct((NC, N_PAD, HID), jnp.float32),
            jax.ShapeDtypeStruct((NC, N_PAD, 16), jnp.float32),
        ],
        mesh=mesh,
        scratch_types=[
            pltpu.VMEM((2, K), jnp.int32),
            pltpu.VMEM((2, K), jnp.int32),
            pltpu.VMEM((K, 16), jnp.float32),
            pltpu.VMEM((K, 16), jnp.float32),
            pltpu.VMEM((K, 16), jnp.float32),
            pltpu.VMEM((K, 16), jnp.float32),
            pltpu.VMEM((K, HID), jnp.float32),
            pltpu.VMEM((K, HID), jnp.float32),
            pltpu.VMEM((K, 16), jnp.float32),
            pltpu.VMEM((K, 16), jnp.float32),
            pltpu.VMEM((K * H,), jnp.float32),
            pltpu.VMEM((ZR, HID), jnp.float32),
            pltpu.VMEM((ZR, 16), jnp.float32),
            pltpu.VMEM_SHARED((N_PAD, HID), jnp.float32),
            pltpu.VMEM_SHARED((N_PAD, 16), jnp.float32),
            pltpu.SemaphoreType.DMA,
            pltpu.SemaphoreType.DMA,
            pltpu.SemaphoreType.DMA,
            pltpu.SemaphoreType.DMA,
            pltpu.SemaphoreType.DMA,
            pltpu.SemaphoreType.DMA,
            pltpu.SemaphoreType.DMA,
            pltpu.SemaphoreType.DMA,
            pltpu.SemaphoreType.DMA,
            pltpu.SemaphoreType.DMA,
        ],
        name="gat_edge_sc",
        compiler_params=pltpu.CompilerParams(use_tc_tiling_on_sc=False,
                                             needs_layout_passes=False),
    )
    return f(idx2, hh, a_all)


def _ln_relu(out, g, b):
    mu = jnp.mean(out, axis=-1, keepdims=True)
    var = jnp.mean((out - mu) ** 2, axis=-1, keepdims=True)
    h = (out - mu) / jnp.sqrt(var + 1e-5) * g + b
    return jnp.maximum(h, 0.0)


def _epilogue(accm_ref, accd_ref, bias_ref, g_ref, b_ref):
    msg = (accm_ref[0] + accm_ref[1]).reshape(256, H, C)
    accd = accd_ref[0] + accd_ref[1]
    den = accd[:, :H]
    out = (msg / (den + 1e-16)[:, :, None]).reshape(256, HID)
    return _ln_relu(out + bias_ref[...], g_ref[...], b_ref[...])


def _embed_body(x_ref, emb_ref, W_ref, A_ref, h_ref, hh_ref, a_ref):
    xb = x_ref[0, 0, :]
    ohT = (xb[None, :] == lax.broadcasted_iota(jnp.int32, (HID, 256), 0))
    h = lax.dot_general(ohT.astype(jnp.float32), emb_ref[...],
                        (((0,), (0,)), ((), ())),
                        preferred_element_type=jnp.float32)
    h_ref[...] = h
    hh = jnp.dot(h, W_ref[...], preferred_element_type=jnp.float32)
    hh_ref[...] = hh
    a_ref[...] = jnp.dot(hh, A_ref[...], preferred_element_type=jnp.float32)


def _embed(xp3, emb_pad, W0, A0):
    return pl.pallas_call(
        _embed_body,
        grid=(NB,),
        in_specs=[
            pl.BlockSpec((1, 1, 256), lambda i: (i, 0, 0)),
            pl.BlockSpec((HID, HID), lambda i: (0, 0)),
            pl.BlockSpec((HID, HID), lambda i: (0, 0)),
            pl.BlockSpec((HID, 16), lambda i: (0, 0)),
        ],
        out_specs=[
            pl.BlockSpec((256, HID), lambda i: (i, 0)),
            pl.BlockSpec((256, HID), lambda i: (i, 0)),
            pl.BlockSpec((256, 16), lambda i: (i, 0)),
        ],
        out_shape=[
            jax.ShapeDtypeStruct((N_PAD, HID), jnp.float32),
            jax.ShapeDtypeStruct((N_PAD, HID), jnp.float32),
            jax.ShapeDtypeStruct((N_PAD, 16), jnp.float32),
        ],
        name="gat_embed_tc",
    )(xp3, emb_pad, W0, A0)


def _mid_body(first, accm_ref, accd_ref, res_ref, bias_ref, g_ref, b_ref,
              W_ref, A_ref, h_ref, hh_ref, a_ref):
    h = _epilogue(accm_ref, accd_ref, bias_ref, g_ref, b_ref)
    if not first:
        h = h + res_ref[...]
    h_ref[...] = h
    hh = jnp.dot(h, W_ref[...], preferred_element_type=jnp.float32)
    hh_ref[...] = hh
    a_ref[...] = jnp.dot(hh, A_ref[...], preferred_element_type=jnp.float32)


def _mid(accm, accd, res, bias, g, b, Wn, An, first):
    return pl.pallas_call(
        functools.partial(_mid_body, first),
        grid=(NB,),
        in_specs=[
            pl.BlockSpec((NC, 256, HID), lambda i: (0, i, 0)),
            pl.BlockSpec((NC, 256, 16), lambda i: (0, i, 0)),
            pl.BlockSpec((256, HID), lambda i: (i, 0)),
            pl.BlockSpec((1, HID), lambda i: (0, 0)),
            pl.BlockSpec((1, HID), lambda i: (0, 0)),
            pl.BlockSpec((1, HID), lambda i: (0, 0)),
            pl.BlockSpec((HID, HID), lambda i: (0, 0)),
            pl.BlockSpec((HID, 16), lambda i: (0, 0)),
        ],
        out_specs=[
            pl.BlockSpec((256, HID), lambda i: (i, 0)),
            pl.BlockSpec((256, HID), lambda i: (i, 0)),
            pl.BlockSpec((256, 16), lambda i: (i, 0)),
        ],
        out_shape=[
            jax.ShapeDtypeStruct((N_PAD, HID), jnp.float32),
            jax.ShapeDtypeStruct((N_PAD, HID), jnp.float32),
            jax.ShapeDtypeStruct((N_PAD, 16), jnp.float32),
        ],
        name="gat_mid_tc",
    )(accm, accd, res, bias, g, b, Wn, An)


def _final_body(accm_ref, accd_ref, res_ref, bias_ref, g_ref, b_ref,
                W1_ref, b1_ref, W2_ref, b2_ref, W3_ref, b3_ref, y_ref):
    h = _epilogue(accm_ref, accd_ref, bias_ref, g_ref, b_ref) + res_ref[...]
    h1 = jnp.maximum(
        jnp.dot(h, W1_ref[...], preferred_element_type=jnp.float32)
        + b1_ref[...], 0.0)
    h2 = jnp.maximum(
        jnp.dot(h1, W2_ref[...], preferred_element_type=jnp.float32)
        + b2_ref[...], 0.0)
    y_ref[...] = (jnp.dot(h2, W3_ref[...], preferred_element_type=jnp.float32)
                  + b3_ref[...])


def _final(accm, accd, res, bias, g, b, W1, b1, W2, b2, W3p, b3p):
    return pl.pallas_call(
        _final_body,
        grid=(NB,),
        in_specs=[
            pl.BlockSpec((NC, 256, HID), lambda i: (0, i, 0)),
            pl.BlockSpec((NC, 256, 16), lambda i: (0, i, 0)),
            pl.BlockSpec((256, HID), lambda i: (i, 0)),
            pl.BlockSpec((1, HID), lambda i: (0, 0)),
            pl.BlockSpec((1, HID), lambda i: (0, 0)),
            pl.BlockSpec((1, HID), lambda i: (0, 0)),
            pl.BlockSpec((HID, 2 * HID), lambda i: (0, 0)),
            pl.BlockSpec((1, 2 * HID), lambda i: (0, 0)),
            pl.BlockSpec((2 * HID, HID), lambda i: (0, 0)),
            pl.BlockSpec((1, HID), lambda i: (0, 0)),
            pl.BlockSpec((HID, HID), lambda i: (0, 0)),
            pl.BlockSpec((1, HID), lambda i: (0, 0)),
        ],
        out_specs=[pl.BlockSpec((256, HID), lambda i: (i, 0))],
        out_shape=[jax.ShapeDtypeStruct((N_PAD, HID), jnp.float32)],
        name="gat_final_tc",
    )(accm, accd, res, bias, g, b, W1, b1, W2, b2, W3p, b3p)[0]


def kernel(x, edge_index, batch, params):
    del batch
    f32 = jnp.float32
    i32 = jnp.int32

    xp = jnp.zeros((N_PAD,), i32).at[:N].set(x.astype(i32))
    xp3 = xp.reshape(NB, 1, 256)

    emb_pad = jnp.zeros((HID, HID), f32).at[:VOCAB].set(params["emb"])

    loop = jnp.arange(N, dtype=i32)
    pad = jnp.full((E_PAD - E_TOT,), DUMMY, i32)
    srcp = jnp.concatenate([edge_index[0].astype(i32), loop, pad])
    dstp = jnp.concatenate([edge_index[1].astype(i32), loop, pad])
    idx2 = jnp.stack([srcp.reshape(NW * NBLK, K),
                      dstp.reshape(NW * NBLK, K)], axis=1)

    rows = np.arange(HID)
    heads = rows // C
    Amats, Ws, biases, gs, bs = [], [], [], [], []
    for p in params["layers"]:
        A = jnp.zeros((HID, 16), f32)
        A = A.at[rows, heads].set(p["att_src"].reshape(HID))
        A = A.at[rows, heads + 8].set(p["att_dst"].reshape(HID))
        Amats.append(A)
        Ws.append(p["W"])
        biases.append(p["bias"].reshape(1, HID))
        gs.append(p["ln_g"].reshape(1, HID))
        bs.append(p["ln_b"].reshape(1, HID))

    h, hh, a_all = _embed(xp3, emb_pad, Ws[0], Amats[0])
    for i in range(NL):
        accm, accd = _sc_edge(hh, a_all, idx2)
        if i < NL - 1:
            h, hh, a_all = _mid(accm, accd, h, biases[i], gs[i], bs[i],
                                Ws[i + 1], Amats[i + 1], first=(i == 0))
        else:
            W3p = jnp.zeros((HID, HID), f32).at[:, :3].set(params["W3"])
            b3p = jnp.zeros((1, HID), f32).at[0, :3].set(params["b3"])
            y = _final(accm, accd, h, biases[i], gs[i], bs[i],
                       params["W1"], params["b1"].reshape(1, 2 * HID),
                       params["W2"], params["b2"].reshape(1, HID),
                       W3p, b3p)
    return y[:N, :3]


# speedup vs baseline: 128.5513x; 1.1571x over previous
import functools

import jax
import jax.numpy as jnp
import numpy as np
from jax import lax
from jax.experimental import pallas as pl
from jax.experimental.pallas import tpu as pltpu
from jax.experimental.pallas import tpu_sc as plsc

N = 10000
HID = 128
H = 8
C = 16
NL = 6
VOCAB = 100

NC, NS, L = 2, 16, 16
NW = NC * NS

N_PAD = 10240
NB = N_PAD // 256
DUMMY = N

E_TOT = 320000 + N
K = 72
EPW = 10368
E_PAD = NW * EPW
NBLK = EPW // K

ACC_W = 144
RPS = N_PAD // NS
ZR = 64


NG = NBLK // 3


def _sc_edge_body(idx_hbm, hh_hbm, aall_hbm, accm_hbm, accd_hbm,
                  idxb0, idxb1, idxb2, sbuf0, dbuf0, sbuf1, dbuf1,
                  sbuf2, dbuf2, hbuf0, hbuf1, hbuf2, wsc0, wsc1, wsc2,
                  accm_sh, accd_sh,
                  sem_h0, sem_s0, sem_d0, sem_m0, sem_w0,
                  sem_h1, sem_s1, sem_d1, sem_m1, sem_w1,
                  sem_h2, sem_s2, sem_d2, sem_m2, sem_w2):
    cid = lax.axis_index("c")
    sid = lax.axis_index("s")
    wid = sid * NC + cid

    def zr(i, _):
        hbuf0[i // (HID // 16), pl.ds((i % (HID // 16)) * 16, 16)] = (
            jnp.zeros((16,), jnp.float32))
        return 0
    lax.fori_loop(0, ZR * (HID // 16), zr, 0)

    def zrd(i, _):
        wsc0[i, pl.ds(0, 16)] = jnp.zeros((16,), jnp.float32)
        return 0
    lax.fori_loop(0, ZR, zrd, 0)

    def zc(k, _):
        row = sid * RPS + k * ZR
        pltpu.sync_copy(hbuf0.at[pl.ds(0, ZR)], accm_sh.at[pl.ds(row, ZR)])
        pltpu.sync_copy(wsc0.at[pl.ds(0, ZR)], accd_sh.at[pl.ds(row, ZR)])
        return 0
    lax.fori_loop(0, RPS // ZR, zc, 0)
    plsc.subcore_barrier()

    lane = lax.iota(jnp.int32, 16)
    halfv = jnp.bitwise_and(lax.shift_right_logical(lane, 3), 1)
    col_s = jnp.bitwise_and(lane, 7)
    col_d = col_s + 8
    izero = jnp.zeros((16,), jnp.int32)
    zero16 = jnp.zeros((16,), jnp.float32)
    below8 = lane < 8
    bbase = wid * NBLK

    bufs = ((idxb0, None, sbuf0, dbuf0, hbuf0, wsc0,
             sem_h0, sem_s0, sem_d0, sem_m0, sem_w0),
            (idxb1, None, sbuf1, dbuf1, hbuf1, wsc1,
             sem_h1, sem_s1, sem_d1, sem_m1, sem_w1),
            (idxb2, None, sbuf2, dbuf2, hbuf2, wsc2,
             sem_h2, sem_s2, sem_d2, sem_m2, sem_w2))

    def idx_load(b, buf):
        pltpu.sync_copy(idx_hbm.at[bbase + b], buf[0])

    def gather_start(buf):
        idxb, sbuf, dbuf, hbuf = buf[0], buf[2], buf[3], buf[4]
        pltpu.async_copy(hh_hbm.at[idxb.at[0]], hbuf, buf[6])
        pltpu.async_copy(aall_hbm.at[idxb.at[0]], sbuf, buf[7])
        pltpu.async_copy(aall_hbm.at[idxb.at[1]], dbuf, buf[8])

    def gather_wait(buf):
        idxb, sbuf, dbuf, hbuf = buf[0], buf[2], buf[3], buf[4]
        pltpu.make_async_copy(hh_hbm.at[idxb.at[0]], hbuf, buf[6]).wait()
        pltpu.make_async_copy(aall_hbm.at[idxb.at[0]], sbuf, buf[7]).wait()
        pltpu.make_async_copy(aall_hbm.at[idxb.at[1]], dbuf, buf[8]).wait()

    def scatter_start(buf):
        idxb, hbuf, wsc = buf[0], buf[4], buf[5]
        pltpu.async_copy(hbuf, accm_sh.at[idxb.at[1]], buf[9], add=True)
        pltpu.async_copy(wsc, accd_sh.at[idxb.at[1]], buf[10], add=True)

    def scatter_wait(buf):
        idxb, hbuf, wsc = buf[0], buf[4], buf[5]
        pltpu.make_async_copy(hbuf, accm_sh.at[idxb.at[1]], buf[9]).wait()
        pltpu.make_async_copy(wsc, accd_sh.at[idxb.at[1]], buf[10]).wait()

    def compute(buf):
        sbuf, dbuf, hbuf, wsc = buf[2], buf[3], buf[4], buf[5]

        @plsc.parallel_loop(0, K, unroll=2)
        def _(r):
            ridx = izero + r
            sv = plsc.load_gather(sbuf, [ridx, col_s])
            dv = plsc.load_gather(dbuf, [ridx, col_d])
            a = sv + dv
            a = jnp.where(a > 0, a, 0.2 * a)
            w = jnp.exp(a)
            wsc[r, pl.ds(0, 16)] = jnp.where(below8, w, zero16)

        @plsc.parallel_loop(0, K, unroll=2)
        def _(r):
            wrow = wsc[r, pl.ds(0, 16)]
            for t in range(H):
                seg = pl.ds(t * C, 16)
                hbuf[r, seg] = hbuf[r, seg] * (zero16 + wrow[t])

    idx_load(0, bufs[0])
    gather_start(bufs[0])
    idx_load(1, bufs[1])
    gather_start(bufs[1])

    def triple(t, _):
        @pl.when(t > 0)
        def _():
            scatter_wait(bufs[2])
        idx_load(3 * t + 2, bufs[2])
        gather_start(bufs[2])
        gather_wait(bufs[0])
        compute(bufs[0])
        scatter_start(bufs[0])
        gather_wait(bufs[1])
        compute(bufs[1])
        scatter_start(bufs[1])
        scatter_wait(bufs[0])
        @pl.when(t < NG - 1)
        def _():
            idx_load(3 * t + 3, bufs[0])
            gather_start(bufs[0])
        gather_wait(bufs[2])
        compute(bufs[2])
        scatter_start(bufs[2])
        scatter_wait(bufs[1])
        @pl.when(t < NG - 1)
        def _():
            idx_load(3 * t + 4, bufs[1])
            gather_start(bufs[1])
        return 0
    lax.fori_loop(0, NG, triple, 0)
    scatter_wait(bufs[2])

    plsc.subcore_barrier()

    def co(k, _):
        row = sid * RPS + k * ZR
        pltpu.sync_copy(accm_sh.at[pl.ds(row, ZR)],
                        accm_hbm.at[cid, pl.ds(row, ZR)])
        pltpu.sync_copy(accd_sh.at[pl.ds(row, ZR)],
                        accd_hbm.at[cid, pl.ds(row, ZR)])
        return 0
    lax.fori_loop(0, RPS // ZR, co, 0)


def _sc_edge(hh, a_all, idx2):
    mesh = plsc.VectorSubcoreMesh(core_axis_name="c", subcore_axis_name="s",
                                  num_cores=NC, num_subcores=NS)
    f = pl.kernel(
        _sc_edge_body,
        out_type=[
            jax.ShapeDtypeStruct((NC, N_PAD, HID), jnp.float32),
            jax.ShapeDtypeStruct((NC, N_PAD, 16), jnp.float32),
        ],
        mesh=mesh,
        scratch_types=[
            pltpu.VMEM((2, K), jnp.int32),
            pltpu.VMEM((2, K), jnp.int32),
            pltpu.VMEM((2, K), jnp.int32),
            pltpu.VMEM((K, 16), jnp.float32),
            pltpu.VMEM((K, 16), jnp.float32),
            pltpu.VMEM((K, 16), jnp.float32),
            pltpu.VMEM((K, 16), jnp.float32),
            pltpu.VMEM((K, 16), jnp.float32),
            pltpu.VMEM((K, 16), jnp.float32),
            pltpu.VMEM((K, HID), jnp.float32),
            pltpu.VMEM((K, HID), jnp.float32),
            pltpu.VMEM((K, HID), jnp.float32),
            pltpu.VMEM((K, 16), jnp.float32),
            pltpu.VMEM((K, 16), jnp.float32),
            pltpu.VMEM((K, 16), jnp.float32),
            pltpu.VMEM_SHARED((N_PAD, HID), jnp.float32),
            pltpu.VMEM_SHARED((N_PAD, 16), jnp.float32),
        ] + [pltpu.SemaphoreType.DMA] * 15,
        name="gat_edge_sc",
        compiler_params=pltpu.CompilerParams(use_tc_tiling_on_sc=False,
                                             needs_layout_passes=False),
    )
    return f(idx2, hh, a_all)


def _ln_relu(out, g, b):
    mu = jnp.mean(out, axis=-1, keepdims=True)
    var = jnp.mean((out - mu) ** 2, axis=-1, keepdims=True)
    h = (out - mu) / jnp.sqrt(var + 1e-5) * g + b
    return jnp.maximum(h, 0.0)


def _epilogue(accm_ref, accd_ref, bias_ref, g_ref, b_ref):
    msg = (accm_ref[0] + accm_ref[1]).reshape(256, H, C)
    accd = accd_ref[0] + accd_ref[1]
    den = accd[:, :H]
    out = (msg / (den + 1e-16)[:, :, None]).reshape(256, HID)
    return _ln_relu(out + bias_ref[...], g_ref[...], b_ref[...])


def _embed_body(x_ref, emb_ref, W_ref, A_ref, h_ref, hh_ref, a_ref):
    xb = x_ref[0, 0, :]
    ohT = (xb[None, :] == lax.broadcasted_iota(jnp.int32, (HID, 256), 0))
    h = lax.dot_general(ohT.astype(jnp.float32), emb_ref[...],
                        (((0,), (0,)), ((), ())),
                        preferred_element_type=jnp.float32)
    h_ref[...] = h
    hh = jnp.dot(h, W_ref[...], preferred_element_type=jnp.float32)
    hh_ref[...] = hh
    a_ref[...] = jnp.dot(hh, A_ref[...], preferred_element_type=jnp.float32)


def _embed(xp3, emb_pad, W0, A0):
    return pl.pallas_call(
        _embed_body,
        grid=(NB,),
        in_specs=[
            pl.BlockSpec((1, 1, 256), lambda i: (i, 0, 0)),
            pl.BlockSpec((HID, HID), lambda i: (0, 0)),
            pl.BlockSpec((HID, HID), lambda i: (0, 0)),
            pl.BlockSpec((HID, 16), lambda i: (0, 0)),
        ],
        out_specs=[
            pl.BlockSpec((256, HID), lambda i: (i, 0)),
            pl.BlockSpec((256, HID), lambda i: (i, 0)),
            pl.BlockSpec((256, 16), lambda i: (i, 0)),
        ],
        out_shape=[
            jax.ShapeDtypeStruct((N_PAD, HID), jnp.float32),
            jax.ShapeDtypeStruct((N_PAD, HID), jnp.float32),
            jax.ShapeDtypeStruct((N_PAD, 16), jnp.float32),
        ],
        name="gat_embed_tc",
    )(xp3, emb_pad, W0, A0)


def _mid_body(first, accm_ref, accd_ref, res_ref, bias_ref, g_ref, b_ref,
              W_ref, A_ref, h_ref, hh_ref, a_ref):
    h = _epilogue(accm_ref, accd_ref, bias_ref, g_ref, b_ref)
    if not first:
        h = h + res_ref[...]
    h_ref[...] = h
    hh = jnp.dot(h, W_ref[...], preferred_element_type=jnp.float32)
    hh_ref[...] = hh
    a_ref[...] = jnp.dot(hh, A_ref[...], preferred_element_type=jnp.float32)


def _mid(accm, accd, res, bias, g, b, Wn, An, first):
    return pl.pallas_call(
        functools.partial(_mid_body, first),
        grid=(NB,),
        in_specs=[
            pl.BlockSpec((NC, 256, HID), lambda i: (0, i, 0)),
            pl.BlockSpec((NC, 256, 16), lambda i: (0, i, 0)),
            pl.BlockSpec((256, HID), lambda i: (i, 0)),
            pl.BlockSpec((1, HID), lambda i: (0, 0)),
            pl.BlockSpec((1, HID), lambda i: (0, 0)),
            pl.BlockSpec((1, HID), lambda i: (0, 0)),
            pl.BlockSpec((HID, HID), lambda i: (0, 0)),
            pl.BlockSpec((HID, 16), lambda i: (0, 0)),
        ],
        out_specs=[
            pl.BlockSpec((256, HID), lambda i: (i, 0)),
            pl.BlockSpec((256, HID), lambda i: (i, 0)),
            pl.BlockSpec((256, 16), lambda i: (i, 0)),
        ],
        out_shape=[
            jax.ShapeDtypeStruct((N_PAD, HID), jnp.float32),
            jax.ShapeDtypeStruct((N_PAD, HID), jnp.float32),
            jax.ShapeDtypeStruct((N_PAD, 16), jnp.float32),
        ],
        name="gat_mid_tc",
    )(accm, accd, res, bias, g, b, Wn, An)


def _final_body(accm_ref, accd_ref, res_ref, bias_ref, g_ref, b_ref,
                W1_ref, b1_ref, W2_ref, b2_ref, W3_ref, b3_ref, y_ref):
    h = _epilogue(accm_ref, accd_ref, bias_ref, g_ref, b_ref) + res_ref[...]
    h1 = jnp.maximum(
        jnp.dot(h, W1_ref[...], preferred_element_type=jnp.float32)
        + b1_ref[...], 0.0)
    h2 = jnp.maximum(
        jnp.dot(h1, W2_ref[...], preferred_element_type=jnp.float32)
        + b2_ref[...], 0.0)
    y_ref[...] = (jnp.dot(h2, W3_ref[...], preferred_element_type=jnp.float32)
                  + b3_ref[...])


def _final(accm, accd, res, bias, g, b, W1, b1, W2, b2, W3p, b3p):
    return pl.pallas_call(
        _final_body,
        grid=(NB,),
        in_specs=[
            pl.BlockSpec((NC, 256, HID), lambda i: (0, i, 0)),
            pl.BlockSpec((NC, 256, 16), lambda i: (0, i, 0)),
            pl.BlockSpec((256, HID), lambda i: (i, 0)),
            pl.BlockSpec((1, HID), lambda i: (0, 0)),
            pl.BlockSpec((1, HID), lambda i: (0, 0)),
            pl.BlockSpec((1, HID), lambda i: (0, 0)),
            pl.BlockSpec((HID, 2 * HID), lambda i: (0, 0)),
            pl.BlockSpec((1, 2 * HID), lambda i: (0, 0)),
            pl.BlockSpec((2 * HID, HID), lambda i: (0, 0)),
            pl.BlockSpec((1, HID), lambda i: (0, 0)),
            pl.BlockSpec((HID, HID), lambda i: (0, 0)),
            pl.BlockSpec((1, HID), lambda i: (0, 0)),
        ],
        out_specs=[pl.BlockSpec((256, HID), lambda i: (i, 0))],
        out_shape=[jax.ShapeDtypeStruct((N_PAD, HID), jnp.float32)],
        name="gat_final_tc",
    )(accm, accd, res, bias, g, b, W1, b1, W2, b2, W3p, b3p)[0]


def kernel(x, edge_index, batch, params):
    del batch
    f32 = jnp.float32
    i32 = jnp.int32

    xp = jnp.zeros((N_PAD,), i32).at[:N].set(x.astype(i32))
    xp3 = xp.reshape(NB, 1, 256)

    emb_pad = jnp.zeros((HID, HID), f32).at[:VOCAB].set(params["emb"])

    loop = jnp.arange(N, dtype=i32)
    pad = jnp.full((E_PAD - E_TOT,), DUMMY, i32)
    srcp = jnp.concatenate([edge_index[0].astype(i32), loop, pad])
    dstp = jnp.concatenate([edge_index[1].astype(i32), loop, pad])
    idx2 = jnp.stack([srcp.reshape(NW * NBLK, K),
                      dstp.reshape(NW * NBLK, K)], axis=1)

    rows = np.arange(HID)
    heads = rows // C
    Amats, Ws, biases, gs, bs = [], [], [], [], []
    for p in params["layers"]:
        A = jnp.zeros((HID, 16), f32)
        A = A.at[rows, heads].set(p["att_src"].reshape(HID))
        A = A.at[rows, heads + 8].set(p["att_dst"].reshape(HID))
        Amats.append(A)
        Ws.append(p["W"])
        biases.append(p["bias"].reshape(1, HID))
        gs.append(p["ln_g"].reshape(1, HID))
        bs.append(p["ln_b"].reshape(1, HID))

    h, hh, a_all = _embed(xp3, emb_pad, Ws[0], Amats[0])
    for i in range(NL):
        accm, accd = _sc_edge(hh, a_all, idx2)
        if i < NL - 1:
            h, hh, a_all = _mid(accm, accd, h, biases[i], gs[i], bs[i],
                                Ws[i + 1], Amats[i + 1], first=(i == 0))
        else:
            W3p = jnp.zeros((HID, HID), f32).at[:, :3].set(params["W3"])
            b3p = jnp.zeros((1, HID), f32).at[0, :3].set(params["b3"])
            y = _final(accm, accd, h, biases[i], gs[i], bs[i],
                       params["W1"], params["b1"].reshape(1, 2 * HID),
                       params["W2"], params["b2"].reshape(1, HID),
                       W3p, b3p)
    return y[:N, :3]
